# Initial kernel scaffold; baseline (speedup 1.0000x reference)
import jax
import jax.numpy as jnp
from jax.experimental import pallas as pl


def kernel(math_hidden_states, segment_ids, W1_matrices, W2_matrices, b1_bias, b2_bias):
    raise NotImplementedError("write your pallas kernel here")


# speedup vs baseline: 11.7193x; 11.7193x over previous
import functools

import jax
import jax.numpy as jnp
from jax.experimental import pallas as pl
from jax.experimental.pallas import tpu as pltpu

T = 2048
H = 768
M = 128
E = 16


def _moe_body(seg_ref, x_ref, w1_ref, w2_ref, b1_ref, b2_ref, out_ref):
    e = pl.program_id(0)

    @pl.when(e == 0)
    def _():
        out_ref[...] = jnp.zeros_like(out_ref)

    x = x_ref[...]
    w1 = w1_ref[0]
    w2 = w2_ref[0]
    hid = jax.lax.dot_general(
        x, w1, (((1,), (1,)), ((), ())),
        preferred_element_type=jnp.float32)
    hid = jnp.maximum(hid + b1_ref[0], 0.0)
    y = jax.lax.dot_general(
        hid, w2, (((1,), (1,)), ((), ())),
        preferred_element_type=jnp.float32)
    y = y + b2_ref[0]
    mask = seg_ref[...] == e
    out_ref[...] += jnp.where(mask, y, 0.0)


@jax.jit
def kernel(math_hidden_states, segment_ids, W1_matrices, W2_matrices, b1_bias, b2_bias):
    seg2d = segment_ids.reshape(T, 1)
    b1_3d = b1_bias.reshape(E, 1, M)
    b2_3d = b2_bias.reshape(E, 1, H)
    grid_spec = pltpu.PrefetchScalarGridSpec(
        num_scalar_prefetch=0,
        grid=(E,),
        in_specs=[
            pl.BlockSpec((T, 1), lambda e: (0, 0)),
            pl.BlockSpec((T, H), lambda e: (0, 0)),
            pl.BlockSpec((1, M, H), lambda e: (e, 0, 0)),
            pl.BlockSpec((1, H, M), lambda e: (e, 0, 0)),
            pl.BlockSpec((1, 1, M), lambda e: (e, 0, 0)),
            pl.BlockSpec((1, 1, H), lambda e: (e, 0, 0)),
        ],
        out_specs=pl.BlockSpec((T, H), lambda e: (0, 0)),
    )
    return pl.pallas_call(
        _moe_body,
        grid_spec=grid_spec,
        out_shape=jax.ShapeDtypeStruct((T, H), jnp.float32),
        compiler_params=pltpu.CompilerParams(
            dimension_semantics=("arbitrary",),
        ),
    )(seg2d, math_hidden_states, W1_matrices, W2_matrices, b1_3d, b2_3d)
